# Initial kernel scaffold; baseline (speedup 1.0000x reference)
import jax
import jax.numpy as jnp
from jax.experimental import pallas as pl


def kernel(x, edge_attr, edge_index, batch, mask, W_node, b_node, W_edge, b_edge, init_u, We1, be1, We2, be2, ge, gbe, Wn1, bn1, Wn2, bn2, gn, gbn, Wu1, bu1, Wu2, bu2, gu, gbu, Wa1, ba1, Wa2, ba2, Wc1, bc1, Wc2, bc2):
    raise NotImplementedError("write your pallas kernel here")


# speedup vs baseline: 1.3507x; 1.3507x over previous
import functools

import jax
import jax.numpy as jnp
from jax import lax
from jax.experimental import pallas as pl
from jax.experimental.pallas import tpu as pltpu

N = 10000
E = 320000
G = 16
NODE_F = 128
H = 64

_INTERPRET = False


def _mm(a, b):
    return lax.dot_general(a, b, (((a.ndim - 1,), (0,)), ((), ())),
                           preferred_element_type=jnp.float32,
                           precision=lax.Precision.HIGHEST)


def _mmT(a, b):
    return lax.dot_general(a, b, (((0,), (0,)), ((), ())),
                           preferred_element_type=jnp.float32,
                           precision=lax.Precision.HIGHEST)


def _ln(t, g, b):
    m = jnp.mean(t, axis=-1, keepdims=True)
    v = jnp.mean((t - m) * (t - m), axis=-1, keepdims=True)
    return (t - m) / jnp.sqrt(v + 1e-5) * g + b


def _iota16():
    return lax.broadcasted_iota(jnp.int32, (1, G), 1)


_NB = 2000
_EB = 8000


def _enc_nodes_body(x_ref, bcol_ref, Wn_ref, bn_ref, A_ref, B_ref, ub1_ref,
                    h_ref, hs_ref, hd_ref, cnt_ref):
    i = pl.program_id(0)
    h = jnp.maximum(_mm(x_ref[...], Wn_ref[...]) + bn_ref[...], 0.0)
    h_ref[...] = h
    hs_ref[...] = _mm(h, A_ref[...]) + ub1_ref[...]
    hd_ref[...] = _mm(h, B_ref[...])
    oh = (bcol_ref[...] == _iota16()).astype(jnp.float32)
    pc = _mmT(oh, jnp.ones((_NB, 1), jnp.float32))

    @pl.when(i == 0)
    def _():
        cnt_ref[...] = pc

    @pl.when(i > 0)
    def _():
        cnt_ref[...] += pc


def _enc_nodes(x, bcol, W_node, b_node, A, B, ub1):
    grid = N // _NB
    out = pl.pallas_call(
        _enc_nodes_body,
        grid=(grid,),
        in_specs=[
            pl.BlockSpec((_NB, NODE_F), lambda i: (i, 0)),
            pl.BlockSpec((_NB, 1), lambda i: (i, 0)),
            pl.BlockSpec((NODE_F, H), lambda i: (0, 0)),
            pl.BlockSpec((1, H), lambda i: (0, 0)),
            pl.BlockSpec((H, H), lambda i: (0, 0)),
            pl.BlockSpec((H, H), lambda i: (0, 0)),
            pl.BlockSpec((1, H), lambda i: (0, 0)),
        ],
        out_specs=[
            pl.BlockSpec((_NB, H), lambda i: (i, 0)),
            pl.BlockSpec((_NB, H), lambda i: (i, 0)),
            pl.BlockSpec((_NB, H), lambda i: (i, 0)),
            pl.BlockSpec((G, 1), lambda i: (0, 0)),
        ],
        out_shape=[
            jax.ShapeDtypeStruct((N, H), jnp.float32),
            jax.ShapeDtypeStruct((N, H), jnp.float32),
            jax.ShapeDtypeStruct((N, H), jnp.float32),
            jax.ShapeDtypeStruct((G, 1), jnp.float32),
        ],
        interpret=_INTERPRET,
    )(x, bcol, W_node, b_node, A, B, ub1)
    return out


def _enc_edges_body(ea_ref, We_ref, be_ref, e_ref):
    e_ref[...] = jnp.maximum(_mm(ea_ref[...], We_ref[...]) + be_ref[...], 0.0)


def _enc_edges(edge_attr, W_edge, b_edge):
    grid = E // _EB
    return pl.pallas_call(
        _enc_edges_body,
        grid=(grid,),
        in_specs=[
            pl.BlockSpec((_EB, 16), lambda i: (i, 0)),
            pl.BlockSpec((16, H), lambda i: (0, 0)),
            pl.BlockSpec((1, H), lambda i: (0, 0)),
        ],
        out_specs=pl.BlockSpec((_EB, H), lambda i: (i, 0)),
        out_shape=jax.ShapeDtypeStruct((E, H), jnp.float32),
        interpret=_INTERPRET,
    )(edge_attr, W_edge, b_edge)


def _nodepre_body(h_ref, bcol_ref, u_ref, A_ref, B_ref, D_ref, be1_ref,
                  hs_ref, hd_ref):
    ub = _mm(u_ref[...], D_ref[...]) + be1_ref[...]
    oh = (bcol_ref[...] == _iota16()).astype(jnp.float32)
    hs_ref[...] = _mm(h_ref[...], A_ref[...]) + _mm(oh, ub)
    hd_ref[...] = _mm(h_ref[...], B_ref[...])


def _nodepre(h, bcol, u, A, B, D, be1):
    grid = N // _NB
    return pl.pallas_call(
        _nodepre_body,
        grid=(grid,),
        in_specs=[
            pl.BlockSpec((_NB, H), lambda i: (i, 0)),
            pl.BlockSpec((_NB, 1), lambda i: (i, 0)),
            pl.BlockSpec((G, H), lambda i: (0, 0)),
            pl.BlockSpec((H, H), lambda i: (0, 0)),
            pl.BlockSpec((H, H), lambda i: (0, 0)),
            pl.BlockSpec((H, H), lambda i: (0, 0)),
            pl.BlockSpec((1, H), lambda i: (0, 0)),
        ],
        out_specs=[
            pl.BlockSpec((_NB, H), lambda i: (i, 0)),
            pl.BlockSpec((_NB, H), lambda i: (i, 0)),
        ],
        out_shape=[
            jax.ShapeDtypeStruct((N, H), jnp.float32),
            jax.ShapeDtypeStruct((N, H), jnp.float32),
        ],
        interpret=_INTERPRET,
    )(h, bcol, u, A, B, D, be1)


def _edge_upd_body(e_ref, g1_ref, g2_ref, scol_ref, lo_ref, hi_ref,
                   C_ref, W2_ref, b2_ref, ge_ref, gbe_ref,
                   eo_ref, esum_ref, ecnt_ref):
    i = pl.program_id(0)
    m = jnp.maximum(_mm(e_ref[...], C_ref[...]) + g1_ref[...] + g2_ref[...],
                    0.0)
    t = _mm(m, W2_ref[...]) + b2_ref[...]
    en = _ln(t, ge_ref[...], gbe_ref[...])
    eo_ref[...] = en
    s = scol_ref[...]
    oh = ((s >= lo_ref[...]) & (s < hi_ref[...])).astype(jnp.float32)
    ps = _mmT(oh, en)
    pc = _mmT(oh, jnp.ones((_EB, 1), jnp.float32))

    @pl.when(i == 0)
    def _():
        esum_ref[...] = ps
        ecnt_ref[...] = pc

    @pl.when(i > 0)
    def _():
        esum_ref[...] += ps
        ecnt_ref[...] += pc


def _edge_update(e, g1, g2, scol, lo, hi, C, We2, be2, ge, gbe):
    grid = E // _EB
    return pl.pallas_call(
        _edge_upd_body,
        grid=(grid,),
        in_specs=[
            pl.BlockSpec((_EB, H), lambda i: (i, 0)),
            pl.BlockSpec((_EB, H), lambda i: (i, 0)),
            pl.BlockSpec((_EB, H), lambda i: (i, 0)),
            pl.BlockSpec((_EB, 1), lambda i: (i, 0)),
            pl.BlockSpec((1, G), lambda i: (0, 0)),
            pl.BlockSpec((1, G), lambda i: (0, 0)),
            pl.BlockSpec((H, H), lambda i: (0, 0)),
            pl.BlockSpec((H, H), lambda i: (0, 0)),
            pl.BlockSpec((1, H), lambda i: (0, 0)),
            pl.BlockSpec((1, H), lambda i: (0, 0)),
            pl.BlockSpec((1, H), lambda i: (0, 0)),
        ],
        out_specs=[
            pl.BlockSpec((_EB, H), lambda i: (i, 0)),
            pl.BlockSpec((G, H), lambda i: (0, 0)),
            pl.BlockSpec((G, 1), lambda i: (0, 0)),
        ],
        out_shape=[
            jax.ShapeDtypeStruct((E, H), jnp.float32),
            jax.ShapeDtypeStruct((G, H), jnp.float32),
            jax.ShapeDtypeStruct((G, 1), jnp.float32),
        ],
        interpret=_INTERPRET,
    )(e, g1, g2, scol, lo, hi, C, We2, be2, ge, gbe)


def _node_upd_body(h_ref, a0_ref, a1_ref, bcol_ref, u_ref,
                   P_ref, Q_ref, R_ref, b1_ref, W2_ref, b2_ref,
                   gn_ref, gbn_ref, ho_ref, nsum_ref):
    i = pl.program_id(0)
    ur = _mm(u_ref[...], R_ref[...]) + b1_ref[...]
    oh = (bcol_ref[...] == _iota16()).astype(jnp.float32)
    agg = a0_ref[...] + a1_ref[...]
    t1 = jnp.maximum(_mm(h_ref[...], P_ref[...]) + _mm(agg, Q_ref[...])
                     + _mm(oh, ur), 0.0)
    t = _mm(t1, W2_ref[...]) + b2_ref[...]
    hn = _ln(t, gn_ref[...], gbn_ref[...])
    ho_ref[...] = hn
    ps = _mmT(oh, hn)

    @pl.when(i == 0)
    def _():
        nsum_ref[...] = ps

    @pl.when(i > 0)
    def _():
        nsum_ref[...] += ps


def _node_update(h, agg0, agg1, bcol, u, P, Q, R, bn1, Wn2, bn2, gn, gbn):
    grid = N // _NB
    return pl.pallas_call(
        _node_upd_body,
        grid=(grid,),
        in_specs=[
            pl.BlockSpec((_NB, H), lambda i: (i, 0)),
            pl.BlockSpec((_NB, H), lambda i: (i, 0)),
            pl.BlockSpec((_NB, H), lambda i: (i, 0)),
            pl.BlockSpec((_NB, 1), lambda i: (i, 0)),
            pl.BlockSpec((G, H), lambda i: (0, 0)),
            pl.BlockSpec((H, H), lambda i: (0, 0)),
            pl.BlockSpec((H, H), lambda i: (0, 0)),
            pl.BlockSpec((H, H), lambda i: (0, 0)),
            pl.BlockSpec((1, H), lambda i: (0, 0)),
            pl.BlockSpec((H, H), lambda i: (0, 0)),
            pl.BlockSpec((1, H), lambda i: (0, 0)),
            pl.BlockSpec((1, H), lambda i: (0, 0)),
            pl.BlockSpec((1, H), lambda i: (0, 0)),
        ],
        out_specs=[
            pl.BlockSpec((_NB, H), lambda i: (i, 0)),
            pl.BlockSpec((G, H), lambda i: (0, 0)),
        ],
        out_shape=[
            jax.ShapeDtypeStruct((N, H), jnp.float32),
            jax.ShapeDtypeStruct((G, H), jnp.float32),
        ],
        interpret=_INTERPRET,
    )(h, agg0, agg1, bcol, u, P, Q, R, bn1, Wn2, bn2, gn, gbn)


def _u_upd_body(last, u_ref, nsum_ref, ncnt_ref, esum_ref, ecnt_ref,
                S_ref, T_ref, V_ref, b1_ref, W2_ref, b2_ref, gu_ref, gbu_ref,
                Wc1_ref, bc1_ref, Wc2_ref, bc2_ref, uo_ref, val_ref):
    nm = nsum_ref[...] / jnp.maximum(ncnt_ref[...], 1.0)
    em = esum_ref[...] / jnp.maximum(ecnt_ref[...], 1.0)
    t1 = jnp.maximum(_mm(u_ref[...], S_ref[...]) + _mm(nm, T_ref[...])
                     + _mm(em, V_ref[...]) + b1_ref[...], 0.0)
    t = _mm(t1, W2_ref[...]) + b2_ref[...]
    un = _ln(t, gu_ref[...], gbu_ref[...])
    uo_ref[...] = un
    if last:
        v1 = jnp.maximum(_mm(un, Wc1_ref[...]) + bc1_ref[...], 0.0)
        val_ref[...] = _mm(v1, Wc2_ref[...]) + bc2_ref[...]
    else:
        val_ref[...] = jnp.zeros((G, 1), jnp.float32)


def _u_update(last, u, nsum, ncnt, esum, ecnt, S, T, V, bu1, Wu2, bu2, gu,
              gbu, Wc1, bc1, Wc2, bc2):
    small = lambda r, c: pl.BlockSpec((r, c), lambda: (0, 0))
    return pl.pallas_call(
        functools.partial(_u_upd_body, last),
        in_specs=[
            small(G, H), small(G, H), small(G, 1), small(G, H), small(G, 1),
            small(H, H), small(H, H), small(H, H), small(1, H),
            small(H, H), small(1, H), small(1, H), small(1, H),
            small(H, H), small(1, H), small(H, 1), small(1, 1),
        ],
        out_specs=[small(G, H), small(G, 1)],
        out_shape=[
            jax.ShapeDtypeStruct((G, H), jnp.float32),
            jax.ShapeDtypeStruct((G, 1), jnp.float32),
        ],
        interpret=_INTERPRET,
    )(u, nsum, ncnt, esum, ecnt, S, T, V, bu1, Wu2, bu2, gu, gbu,
      Wc1, bc1, Wc2, bc2)


_NG = N // G


def _final_body(h3_ref, m3_ref, Wa1_ref, ba1_ref, Wa2_ref, ba2_ref,
                act_ref, lp_ref, ent_ref):
    acts, lps, ents = [], [], []
    for k in range(G):
        hk = h3_ref[k]
        s1 = jnp.maximum(_mm(hk, Wa1_ref[...]) + ba1_ref[...], 0.0)
        s = _mm(s1, Wa2_ref[...]) + ba2_ref[...]
        logit = jnp.where(m3_ref[k] > 0.0, s, jnp.float32(-1e9))
        mx = jnp.max(logit, axis=0, keepdims=True)
        sh = logit - mx
        ex = jnp.exp(sh)
        sm = jnp.sum(ex, axis=0, keepdims=True)
        logz = jnp.log(sm)
        logp = sh - logz
        ridx = lax.broadcasted_iota(jnp.int32, (_NG, 1), 0)
        cand = jnp.where(logit == mx, ridx, jnp.int32(2 ** 30))
        act = jnp.min(cand, axis=0, keepdims=True)
        p = ex / sm
        ent = -jnp.sum(p * logp, axis=0, keepdims=True)
        acts.append(act)
        lps.append(-logz)
        ents.append(ent)
    act_ref[...] = jnp.concatenate(acts, axis=0)
    lp_ref[...] = jnp.concatenate(lps, axis=0)
    ent_ref[...] = jnp.concatenate(ents, axis=0)


def _final(h3, m3, Wa1, ba1, Wa2, ba2):
    return pl.pallas_call(
        _final_body,
        in_specs=[
            pl.BlockSpec((G, _NG, H), lambda: (0, 0, 0)),
            pl.BlockSpec((G, _NG, 1), lambda: (0, 0, 0)),
            pl.BlockSpec((H, H), lambda: (0, 0)),
            pl.BlockSpec((1, H), lambda: (0, 0)),
            pl.BlockSpec((H, 1), lambda: (0, 0)),
            pl.BlockSpec((1, 1), lambda: (0, 0)),
        ],
        out_specs=[
            pl.BlockSpec((G, 1), lambda: (0, 0)),
            pl.BlockSpec((G, 1), lambda: (0, 0)),
            pl.BlockSpec((G, 1), lambda: (0, 0)),
        ],
        out_shape=[
            jax.ShapeDtypeStruct((G, 1), jnp.int32),
            jax.ShapeDtypeStruct((G, 1), jnp.float32),
            jax.ShapeDtypeStruct((G, 1), jnp.float32),
        ],
        interpret=_INTERPRET,
    )(h3, m3, Wa1, ba1, Wa2, ba2)


def _gather2(hs, hd, src, dst):
    return jnp.take(hs, src, axis=0), jnp.take(hd, dst, axis=0)


def _scatter_agg(e, dst):
    agg = jax.ops.segment_sum(e, dst, num_segments=N)
    return agg, jnp.zeros_like(agg)


def kernel(x, edge_attr, edge_index, batch, mask, W_node, b_node, W_edge,
           b_edge, init_u, We1, be1, We2, be2, ge, gbe, Wn1, bn1, Wn2, bn2,
           gn, gbn, Wu1, bu1, Wu2, bu2, gu, gbu, Wa1, ba1, Wa2, ba2, Wc1,
           bc1, Wc2, bc2):
    f32 = jnp.float32
    src = edge_index[0]
    dst = edge_index[1]
    bcol = batch[:, None]
    scol = src[:, None]
    row = lambda v: v.reshape(1, -1).astype(f32)
    be1r, be2r, ger, gber = row(be1), row(be2), row(ge), row(gbe)
    bn1r, bn2r, gnr, gbnr = row(bn1), row(bn2), row(gn), row(gbn)
    bu1r, bu2r, gur, gbur = row(bu1), row(bu2), row(gu), row(gbu)
    ba1r, ba2r = row(ba1), row(ba2)
    bnoder, bedger, bc1r, bc2r = row(b_node), row(b_edge), row(bc1), row(bc2)

    A1, B1, C1, D1 = We1[0:H], We1[H:2 * H], We1[2 * H:3 * H], We1[3 * H:]
    P, Q, R = Wn1[0:H], Wn1[H:2 * H], Wn1[2 * H:]
    S, T, V = Wu1[0:H], Wu1[H:2 * H], Wu1[2 * H:]

    ub1 = _mm(init_u.astype(f32), D1) + be1r

    h, hs, hd, ncnt = _enc_nodes(x, bcol, W_node, bnoder, A1, B1, ub1)
    e = _enc_edges(edge_attr, W_edge, bedger)

    cum = jnp.cumsum(ncnt[:, 0].astype(jnp.int32))
    lo = jnp.concatenate([jnp.zeros((1,), jnp.int32), cum[:-1]]).reshape(1, G)
    hi = cum.reshape(1, G)

    u = jnp.broadcast_to(init_u.astype(f32), (G, H))

    for layer in range(2):
        if layer > 0:
            hs, hd = _nodepre(h, bcol, u, A1, B1, D1, be1r)
        g1, g2 = _gather2(hs, hd, src, dst)
        e, esum, ecnt = _edge_update(e, g1, g2, scol, lo, hi, C1, We2, be2r,
                                     ger, gber)
        agg0, agg1 = _scatter_agg(e, dst)
        h, nsum = _node_update(h, agg0, agg1, bcol, u, P, Q, R, bn1r, Wn2,
                               bn2r, gnr, gbnr)
        u, value = _u_update(layer == 1, u, nsum, ncnt, esum, ecnt, S, T, V,
                             bu1r, Wu2, bu2r, gur, gbur, Wc1, bc1r, Wc2,
                             bc2r)

    h3 = h.reshape(G, _NG, H)
    m3 = mask.astype(f32).reshape(G, _NG, 1)
    act, lp, ent = _final(h3, m3, Wa1, ba1r, Wa2, ba2r)
    return (act[:, 0], lp[:, 0], ent[:, 0], value)
